# baseline (device time: 248314 ns/iter reference)
import jax
import jax.numpy as jnp
from jax import lax
from jax.experimental import pallas as pl
from jax.experimental.pallas import tpu as pltpu

N_CHUNKS = 16
PREF = 4


def kernel(x):
    x = x.astype(jnp.bfloat16)
    m, n = x.shape
    rows = m // N_CHUNKS

    def body(x_ref, out_ref, vx,
             lsem_x, send_x, recv_x, send_y, recv_y):
        my_x = lax.axis_index("x")
        my_y = lax.axis_index("y")
        x_peer = (1 - my_x, my_y)
        y_peer = (my_x, 1 - my_y)

        barrier = pltpu.get_barrier_semaphore()
        for nbr in (x_peer, y_peer):
            pl.semaphore_signal(
                barrier, inc=1, device_id=nbr,
                device_id_type=pl.DeviceIdType.MESH,
            )
        pl.semaphore_wait(barrier, 2)

        x_rdmas = []
        for k in range(N_CHUNKS):
            r = pltpu.make_async_remote_copy(
                src_ref=x_ref.at[pl.ds(k * rows, rows), :],
                dst_ref=out_ref.at[pl.ds(k * rows, rows), pl.ds(my_y * n, n)],
                send_sem=send_x.at[k],
                recv_sem=recv_x.at[k],
                device_id=x_peer,
                device_id_type=pl.DeviceIdType.MESH,
            )
            r.start()
            x_rdmas.append(r)

        def load_mine(k):
            cp = pltpu.make_async_copy(
                x_ref.at[pl.ds(k * rows, rows), :],
                vx.at[k % PREF],
                lsem_x.at[k % PREF],
            )
            cp.start()
            return cp

        loads = {}
        for k in range(PREF):
            loads[k] = load_mine(k)

        y_rdmas = []
        for k in range(N_CHUNKS):
            x_rdmas[k].wait_recv()
            loads[k].wait()
            out_ref[pl.ds(k * rows, rows), pl.ds(my_y * n, n)] = (
                out_ref[pl.ds(k * rows, rows), pl.ds(my_y * n, n)]
                + vx[k % PREF, :, :]
            )
            ry = pltpu.make_async_remote_copy(
                src_ref=out_ref.at[pl.ds(k * rows, rows), pl.ds(my_y * n, n)],
                dst_ref=out_ref.at[pl.ds(k * rows, rows), pl.ds(my_y * n, n)],
                send_sem=send_y.at[k],
                recv_sem=recv_y.at[k],
                device_id=y_peer,
                device_id_type=pl.DeviceIdType.MESH,
            )
            ry.start()
            y_rdmas.append(ry)
            if k + PREF < N_CHUNKS:
                loads[k + PREF] = load_mine(k + PREF)

        for k in range(N_CHUNKS):
            x_rdmas[k].wait_send()
            y_rdmas[k].wait_send()
            y_rdmas[k].wait_recv()

    return pl.pallas_call(
        body,
        out_shape=jax.ShapeDtypeStruct((m, 2 * n), jnp.bfloat16),
        in_specs=[pl.BlockSpec(memory_space=pl.ANY)],
        out_specs=pl.BlockSpec(memory_space=pltpu.VMEM),
        scratch_shapes=[
            pltpu.VMEM((PREF, rows, n), jnp.bfloat16),
            pltpu.SemaphoreType.DMA((PREF,)),
            pltpu.SemaphoreType.DMA((N_CHUNKS,)),
            pltpu.SemaphoreType.DMA((N_CHUNKS,)),
            pltpu.SemaphoreType.DMA((N_CHUNKS,)),
            pltpu.SemaphoreType.DMA((N_CHUNKS,)),
        ],
        compiler_params=pltpu.CompilerParams(
            collective_id=0,
            vmem_limit_bytes=100 * 1024 * 1024,
        ),
    )(x)


# device time: 242572 ns/iter; 1.0237x vs baseline; 1.0237x over previous
import jax
import jax.numpy as jnp
from jax import lax
from jax.experimental import pallas as pl
from jax.experimental.pallas import tpu as pltpu

N_CHUNKS = 16


def kernel(x):
    x = x.astype(jnp.bfloat16)
    m, n = x.shape
    rows = m // N_CHUNKS

    def body(x_ref, out_ref, xrecv,
             lsem_o, send_x, recv_x, send_y, recv_y):
        my_x = lax.axis_index("x")
        my_y = lax.axis_index("y")
        x_peer = (1 - my_x, my_y)
        y_peer = (my_x, 1 - my_y)

        barrier = pltpu.get_barrier_semaphore()
        for nbr in (x_peer, y_peer):
            pl.semaphore_signal(
                barrier, inc=1, device_id=nbr,
                device_id_type=pl.DeviceIdType.MESH,
            )
        pl.semaphore_wait(barrier, 2)

        x_rdmas = []
        for k in range(N_CHUNKS):
            r = pltpu.make_async_remote_copy(
                src_ref=x_ref.at[pl.ds(k * rows, rows), :],
                dst_ref=xrecv.at[k],
                send_sem=send_x.at[k],
                recv_sem=recv_x.at[k],
                device_id=x_peer,
                device_id_type=pl.DeviceIdType.MESH,
            )
            r.start()
            x_rdmas.append(r)

        y_rdmas = []
        store_cps = []
        for k in range(N_CHUNKS):
            x_rdmas[k].wait_recv()
            xrecv[k, :, :] = xrecv[k, :, :] + x_ref[pl.ds(k * rows, rows), :]
            ry = pltpu.make_async_remote_copy(
                src_ref=xrecv.at[k],
                dst_ref=out_ref.at[pl.ds(k * rows, rows), pl.ds(my_y * n, n)],
                send_sem=send_y.at[k],
                recv_sem=recv_y.at[k],
                device_id=y_peer,
                device_id_type=pl.DeviceIdType.MESH,
            )
            ry.start()
            y_rdmas.append(ry)
            cp_o = pltpu.make_async_copy(
                xrecv.at[k],
                out_ref.at[pl.ds(k * rows, rows), pl.ds(my_y * n, n)],
                lsem_o.at[k],
            )
            cp_o.start()
            store_cps.append(cp_o)

        for k in range(N_CHUNKS):
            x_rdmas[k].wait_send()
            y_rdmas[k].wait_send()
            y_rdmas[k].wait_recv()
            store_cps[k].wait()

    return pl.pallas_call(
        body,
        out_shape=jax.ShapeDtypeStruct((m, 2 * n), jnp.bfloat16),
        in_specs=[pl.BlockSpec(memory_space=pltpu.VMEM)],
        out_specs=pl.BlockSpec(memory_space=pl.ANY),
        scratch_shapes=[
            pltpu.VMEM((N_CHUNKS, rows, n), jnp.bfloat16),
            pltpu.SemaphoreType.DMA((N_CHUNKS,)),
            pltpu.SemaphoreType.DMA((N_CHUNKS,)),
            pltpu.SemaphoreType.DMA((N_CHUNKS,)),
            pltpu.SemaphoreType.DMA((N_CHUNKS,)),
            pltpu.SemaphoreType.DMA((N_CHUNKS,)),
        ],
        compiler_params=pltpu.CompilerParams(
            collective_id=0,
            vmem_limit_bytes=100 * 1024 * 1024,
        ),
    )(x)


# device time: 226502 ns/iter; 1.0963x vs baseline; 1.0709x over previous
import jax
import jax.numpy as jnp
from jax import lax
from jax.experimental import pallas as pl
from jax.experimental.pallas import tpu as pltpu

N_CHUNKS = 32
PREF = 4


def kernel(x):
    x = x.astype(jnp.bfloat16)
    m, n = x.shape
    rows = m // N_CHUNKS

    def body(x_ref, out_ref, xrecv, vx,
             lsem_x, lsem_o, send_x, recv_x, send_y, recv_y):
        my_x = lax.axis_index("x")
        my_y = lax.axis_index("y")
        x_peer = (1 - my_x, my_y)
        y_peer = (my_x, 1 - my_y)

        barrier = pltpu.get_barrier_semaphore()
        for nbr in (x_peer, y_peer):
            pl.semaphore_signal(
                barrier, inc=1, device_id=nbr,
                device_id_type=pl.DeviceIdType.MESH,
            )
        pl.semaphore_wait(barrier, 2)

        x_rdmas = []
        for k in range(N_CHUNKS):
            r = pltpu.make_async_remote_copy(
                src_ref=x_ref.at[pl.ds(k * rows, rows), :],
                dst_ref=xrecv.at[k],
                send_sem=send_x.at[k],
                recv_sem=recv_x.at[k],
                device_id=x_peer,
                device_id_type=pl.DeviceIdType.MESH,
            )
            r.start()
            x_rdmas.append(r)

        def load_mine(k):
            cp = pltpu.make_async_copy(
                x_ref.at[pl.ds(k * rows, rows), :],
                vx.at[k % PREF],
                lsem_x.at[k % PREF],
            )
            cp.start()
            return cp

        loads = {}
        for k in range(PREF):
            loads[k] = load_mine(k)

        y_rdmas = []
        store_cps = []
        for k in range(N_CHUNKS):
            x_rdmas[k].wait_recv()
            loads[k].wait()
            xrecv[k, :, :] = xrecv[k, :, :] + vx[k % PREF, :, :]
            ry = pltpu.make_async_remote_copy(
                src_ref=xrecv.at[k],
                dst_ref=out_ref.at[pl.ds(k * rows, rows), pl.ds(my_y * n, n)],
                send_sem=send_y.at[k],
                recv_sem=recv_y.at[k],
                device_id=y_peer,
                device_id_type=pl.DeviceIdType.MESH,
            )
            ry.start()
            y_rdmas.append(ry)
            cp_o = pltpu.make_async_copy(
                xrecv.at[k],
                out_ref.at[pl.ds(k * rows, rows), pl.ds(my_y * n, n)],
                lsem_o.at[k],
            )
            cp_o.start()
            store_cps.append(cp_o)
            if k + PREF < N_CHUNKS:
                loads[k + PREF] = load_mine(k + PREF)

        for k in range(N_CHUNKS):
            x_rdmas[k].wait_send()
            y_rdmas[k].wait_send()
            y_rdmas[k].wait_recv()
            store_cps[k].wait()

    return pl.pallas_call(
        body,
        out_shape=jax.ShapeDtypeStruct((m, 2 * n), jnp.bfloat16),
        in_specs=[pl.BlockSpec(memory_space=pl.ANY)],
        out_specs=pl.BlockSpec(memory_space=pl.ANY),
        scratch_shapes=[
            pltpu.VMEM((N_CHUNKS, rows, n), jnp.bfloat16),
            pltpu.VMEM((PREF, rows, n), jnp.bfloat16),
            pltpu.SemaphoreType.DMA((PREF,)),
            pltpu.SemaphoreType.DMA((N_CHUNKS,)),
            pltpu.SemaphoreType.DMA((N_CHUNKS,)),
            pltpu.SemaphoreType.DMA((N_CHUNKS,)),
            pltpu.SemaphoreType.DMA((N_CHUNKS,)),
            pltpu.SemaphoreType.DMA((N_CHUNKS,)),
        ],
        compiler_params=pltpu.CompilerParams(collective_id=0),
    )(x)


# device time: 225107 ns/iter; 1.1031x vs baseline; 1.0062x over previous
import jax
import jax.numpy as jnp
from jax import lax
from jax.experimental import pallas as pl
from jax.experimental.pallas import tpu as pltpu

N_CHUNKS = 64
WINDOW = 8
PREF = 4


def kernel(x):
    x = x.astype(jnp.bfloat16)
    m, n = x.shape
    rows = m // N_CHUNKS

    def body(x_ref, out_ref, xrecv, vx,
             lsem_x, lsem_o, send_x, recv_x, send_y, recv_y):
        my_x = lax.axis_index("x")
        my_y = lax.axis_index("y")
        x_peer = (1 - my_x, my_y)
        y_peer = (my_x, 1 - my_y)

        barrier = pltpu.get_barrier_semaphore()
        for nbr in (x_peer, y_peer):
            pl.semaphore_signal(
                barrier, inc=1, device_id=nbr,
                device_id_type=pl.DeviceIdType.MESH,
            )
        pl.semaphore_wait(barrier, 2)

        def send_chunk(k):
            r = pltpu.make_async_remote_copy(
                src_ref=x_ref.at[pl.ds(k * rows, rows), :],
                dst_ref=xrecv.at[k],
                send_sem=send_x.at[k],
                recv_sem=recv_x.at[k],
                device_id=x_peer,
                device_id_type=pl.DeviceIdType.MESH,
            )
            r.start()
            return r

        def load_mine(k):
            cp = pltpu.make_async_copy(
                x_ref.at[pl.ds(k * rows, rows), :],
                vx.at[k % PREF],
                lsem_x.at[k % PREF],
            )
            cp.start()
            return cp

        x_rdmas = {}
        for k in range(WINDOW):
            x_rdmas[k] = send_chunk(k)
        loads = {}
        for k in range(PREF):
            loads[k] = load_mine(k)

        y_rdmas = []
        store_cps = []
        for k in range(N_CHUNKS):
            x_rdmas[k].wait_recv()
            loads[k].wait()
            xrecv[k, :, :] = xrecv[k, :, :] + vx[k % PREF, :, :]
            ry = pltpu.make_async_remote_copy(
                src_ref=xrecv.at[k],
                dst_ref=out_ref.at[pl.ds(k * rows, rows), pl.ds(my_y * n, n)],
                send_sem=send_y.at[k],
                recv_sem=recv_y.at[k],
                device_id=y_peer,
                device_id_type=pl.DeviceIdType.MESH,
            )
            ry.start()
            y_rdmas.append(ry)
            cp_o = pltpu.make_async_copy(
                xrecv.at[k],
                out_ref.at[pl.ds(k * rows, rows), pl.ds(my_y * n, n)],
                lsem_o.at[k],
            )
            cp_o.start()
            store_cps.append(cp_o)
            if k + WINDOW < N_CHUNKS:
                x_rdmas[k + WINDOW] = send_chunk(k + WINDOW)
            if k + PREF < N_CHUNKS:
                loads[k + PREF] = load_mine(k + PREF)

        for k in range(N_CHUNKS):
            x_rdmas[k].wait_send()
            y_rdmas[k].wait_send()
            y_rdmas[k].wait_recv()
            store_cps[k].wait()

    return pl.pallas_call(
        body,
        out_shape=jax.ShapeDtypeStruct((m, 2 * n), jnp.bfloat16),
        in_specs=[pl.BlockSpec(memory_space=pl.ANY)],
        out_specs=pl.BlockSpec(memory_space=pl.ANY),
        scratch_shapes=[
            pltpu.VMEM((N_CHUNKS, rows, n), jnp.bfloat16),
            pltpu.VMEM((PREF, rows, n), jnp.bfloat16),
            pltpu.SemaphoreType.DMA((PREF,)),
            pltpu.SemaphoreType.DMA((N_CHUNKS,)),
            pltpu.SemaphoreType.DMA((N_CHUNKS,)),
            pltpu.SemaphoreType.DMA((N_CHUNKS,)),
            pltpu.SemaphoreType.DMA((N_CHUNKS,)),
            pltpu.SemaphoreType.DMA((N_CHUNKS,)),
        ],
        compiler_params=pltpu.CompilerParams(collective_id=0),
    )(x)
